# Initial kernel scaffold; baseline (speedup 1.0000x reference)
import jax
import jax.numpy as jnp
from jax.experimental import pallas as pl


def kernel(feat, edge_index, W1, b1, W2, b2, W3, b3):
    raise NotImplementedError("write your pallas kernel here")


# speedup vs baseline: 7.8517x; 7.8517x over previous
import functools

import jax
import jax.numpy as jnp
from jax import lax
from jax.experimental import pallas as pl
from jax.experimental.pallas import tpu as pltpu
from jax.experimental.pallas import tpu_sc as plsc

NC = 2
NS = 16
NW = NC * NS
LANES = 16
CH = 80


def _sc_degree(N_, E_):
    EPW = E_ // NW
    NCH = EPW // CH
    RPS = N_ // NS
    ZR = 125
    mesh = plsc.VectorSubcoreMesh(core_axis_name="c", subcore_axis_name="s")

    @functools.partial(
        pl.kernel,
        out_type=[jax.ShapeDtypeStruct((NC, N_, LANES), jnp.float32),
                  jax.ShapeDtypeStruct((NC, N_, LANES), jnp.float32)],
        mesh=mesh,
        compiler_params=pltpu.CompilerParams(use_tc_tiling_on_sc=False),
        scratch_types=[
            pltpu.VMEM((NCH, CH), jnp.int32),
            pltpu.VMEM((NCH, CH), jnp.int32),
            pltpu.VMEM((CH, LANES), jnp.float32),
            pltpu.VMEM((ZR, LANES), jnp.float32),
            pltpu.VMEM_SHARED((N_, LANES), jnp.float32),
            pltpu.VMEM_SHARED((N_, LANES), jnp.float32),
        ],
    )
    def deg_kernel(src_hbm, dst_hbm, osrc_hbm, odst_hbm,
                   srcv, dstv, ones_v, zeros_v, acc_s, acc_d):
        cid = lax.axis_index("c")
        sid = lax.axis_index("s")
        wid = sid * NC + cid

        @pl.loop(0, CH)
        def _(i):
            ones_v[i, :] = jnp.ones((LANES,), jnp.float32)

        @pl.loop(0, ZR)
        def _(i):
            zeros_v[i, :] = jnp.zeros((LANES,), jnp.float32)

        @pl.loop(0, RPS // ZR)
        def _(i):
            pltpu.sync_copy(zeros_v, acc_s.at[pl.ds(sid * RPS + i * ZR, ZR)])
            pltpu.sync_copy(zeros_v, acc_d.at[pl.ds(sid * RPS + i * ZR, ZR)])

        plsc.subcore_barrier()

        pltpu.sync_copy(src_hbm.at[wid], srcv)
        pltpu.sync_copy(dst_hbm.at[wid], dstv)

        @pl.loop(0, NCH)
        def _(j):
            pltpu.sync_copy(ones_v, acc_s.at[srcv.at[j]], add=True)
            pltpu.sync_copy(ones_v, acc_d.at[dstv.at[j]], add=True)

        plsc.subcore_barrier()

        @pl.when(sid == 0)
        def _():
            pltpu.sync_copy(acc_s, osrc_hbm.at[cid])

        @pl.when(sid == 1)
        def _():
            pltpu.sync_copy(acc_d, odst_hbm.at[cid])

    return deg_kernel


def _sc_aggregate(N_, E_, Dh):
    EPS = E_ // NS
    NCH = EPS // CH
    RPS = N_ // NS
    ZR = 25
    mesh = plsc.VectorSubcoreMesh(core_axis_name="c", subcore_axis_name="s")

    @functools.partial(
        pl.kernel,
        out_type=jax.ShapeDtypeStruct((NC, N_, Dh), jnp.float32),
        mesh=mesh,
        compiler_params=pltpu.CompilerParams(use_tc_tiling_on_sc=False),
        scratch_types=[
            pltpu.VMEM((NCH, CH), jnp.int32),
            pltpu.VMEM((NCH, CH), jnp.int32),
            pltpu.VMEM((CH, Dh), jnp.float32),
            pltpu.VMEM_SHARED((N_, Dh), jnp.float32),
            pltpu.SemaphoreType.DMA,
        ],
    )
    def agg_kernel(h_hbm, src_hbm, dst_hbm, out_hbm,
                   srcv, dstv, rows_v, acc, gsem):
        cid = lax.axis_index("c")
        sid = lax.axis_index("s")

        @pl.loop(0, ZR)
        def _(i):
            @pl.loop(0, Dh // LANES)
            def _(k):
                rows_v[i, pl.ds(k * LANES, LANES)] = jnp.zeros((LANES,), jnp.float32)

        @pl.loop(0, RPS // ZR)
        def _(i):
            pltpu.sync_copy(rows_v.at[pl.ds(0, ZR)],
                            acc.at[pl.ds(sid * RPS + i * ZR, ZR)])

        plsc.subcore_barrier()

        pltpu.sync_copy(src_hbm.at[sid], srcv)
        pltpu.sync_copy(dst_hbm.at[sid], dstv)

        @pl.loop(0, NCH)
        def _(j):
            pltpu.async_copy(h_hbm.at[cid].at[srcv.at[j]], rows_v, gsem).wait()
            pltpu.sync_copy(rows_v, acc.at[dstv.at[j]], add=True)

        plsc.subcore_barrier()

        @pl.when(sid == 0)
        def _():
            pltpu.sync_copy(acc, out_hbm.at[cid])

    return agg_kernel


def _norm(c0, c1):
    deg = c0[:, :1] + c1[:, :1]
    return lax.rsqrt(jnp.maximum(deg, 1.0))


def _tc_pre(feat, W, ds0, ds1, R=1000):
    N_, D = feat.shape
    Dout = W.shape[1]
    Dh = Dout // 2

    def body(f_r, w_r, d0_r, d1_r, o_r):
        h = jnp.dot(f_r[...], w_r[...], preferred_element_type=jnp.float32)
        h = h * _norm(d0_r, d1_r)
        o_r[0] = h[:, :Dh]
        o_r[1] = h[:, Dh:]

    return pl.pallas_call(
        body,
        grid=(N_ // R,),
        in_specs=[pl.BlockSpec((R, D), lambda i: (i, 0)),
                  pl.BlockSpec((D, Dout), lambda i: (0, 0)),
                  pl.BlockSpec((R, LANES), lambda i: (i, 0)),
                  pl.BlockSpec((R, LANES), lambda i: (i, 0))],
        out_specs=pl.BlockSpec((2, R, Dh), lambda i: (0, i, 0)),
        out_shape=jax.ShapeDtypeStruct((2, N_, Dh), jnp.float32),
    )(feat, W, ds0, ds1)


def _tc_mid(p, dd0, dd1, ds0, ds1, b, W, R=1000):
    _, N_, Dh_in = p.shape
    D = 2 * Dh_in
    Dout = W.shape[1]
    Dh = Dout // 2
    b2d = b.reshape(1, D)

    def body(p_r, dd0_r, dd1_r, ds0_r, ds1_r, b_r, w_r, o_r):
        agg = jnp.concatenate([p_r[0], p_r[1]], axis=1)
        h = agg * _norm(dd0_r, dd1_r) + b_r[...]
        h = jnp.maximum(h, 0.0)
        hw = jnp.dot(h, w_r[...], preferred_element_type=jnp.float32)
        hw = hw * _norm(ds0_r, ds1_r)
        o_r[0] = hw[:, :Dh]
        o_r[1] = hw[:, Dh:]

    return pl.pallas_call(
        body,
        grid=(N_ // R,),
        in_specs=[pl.BlockSpec((2, R, Dh_in), lambda i: (0, i, 0)),
                  pl.BlockSpec((R, LANES), lambda i: (i, 0)),
                  pl.BlockSpec((R, LANES), lambda i: (i, 0)),
                  pl.BlockSpec((R, LANES), lambda i: (i, 0)),
                  pl.BlockSpec((R, LANES), lambda i: (i, 0)),
                  pl.BlockSpec((1, D), lambda i: (0, 0)),
                  pl.BlockSpec((D, Dout), lambda i: (0, 0))],
        out_specs=pl.BlockSpec((2, R, Dh), lambda i: (0, i, 0)),
        out_shape=jax.ShapeDtypeStruct((2, N_, Dh), jnp.float32),
    )(p, dd0, dd1, ds0, ds1, b2d, W)


def _tc_post(p, dd0, dd1, b, R=1000):
    _, N_, Dh_in = p.shape
    D = 2 * Dh_in
    b2d = b.reshape(1, D)

    def body(p_r, dd0_r, dd1_r, b_r, o_r):
        agg = jnp.concatenate([p_r[0], p_r[1]], axis=1)
        o_r[...] = agg * _norm(dd0_r, dd1_r) + b_r[...]

    return pl.pallas_call(
        body,
        grid=(N_ // R,),
        in_specs=[pl.BlockSpec((2, R, Dh_in), lambda i: (0, i, 0)),
                  pl.BlockSpec((R, LANES), lambda i: (i, 0)),
                  pl.BlockSpec((R, LANES), lambda i: (i, 0)),
                  pl.BlockSpec((1, D), lambda i: (0, 0))],
        out_specs=pl.BlockSpec((R, D), lambda i: (i, 0)),
        out_shape=jax.ShapeDtypeStruct((N_, D), jnp.float32),
    )(p, dd0, dd1, b2d)


def kernel(feat, edge_index, W1, b1, W2, b2, W3, b3):
    N_, D_in = feat.shape
    E_ = edge_index.shape[1]
    HID_ = W2.shape[1]
    NCLS = W3.shape[1]
    D3 = 64

    src_w = edge_index[0].reshape(NW, (E_ // NW) // CH, CH)
    dst_w = edge_index[1].reshape(NW, (E_ // NW) // CH, CH)
    src_s = edge_index[0].reshape(NS, (E_ // NS) // CH, CH)
    dst_s = edge_index[1].reshape(NS, (E_ // NS) // CH, CH)

    W3p = jnp.zeros((HID_, D3), jnp.float32).at[:, :NCLS].set(W3)
    b3p = jnp.zeros((D3,), jnp.float32).at[:NCLS].set(b3)

    degs, degd = _sc_degree(N_, E_)(src_w, dst_w)
    ds0, ds1 = degs[0], degs[1]
    dd0, dd1 = degd[0], degd[1]

    agg_wide = _sc_aggregate(N_, E_, HID_ // 2)
    agg_narrow = _sc_aggregate(N_, E_, D3 // 2)

    h1 = _tc_pre(feat, W1, ds0, ds1)
    p1 = agg_wide(h1, src_s, dst_s)
    h2 = _tc_mid(p1, dd0, dd1, ds0, ds1, b1, W2)
    p2 = agg_wide(h2, src_s, dst_s)
    h3 = _tc_mid(p2, dd0, dd1, ds0, ds1, b2, W3p)
    p3 = agg_narrow(h3, src_s, dst_s)
    out = _tc_post(p3, dd0, dd1, b3p)
    return out[:, :NCLS]


# speedup vs baseline: 15.7630x; 2.0076x over previous
import functools

import jax
import jax.numpy as jnp
from jax import lax
from jax.experimental import pallas as pl
from jax.experimental.pallas import tpu as pltpu
from jax.experimental.pallas import tpu_sc as plsc

NC = 2
NS = 16
NW = NC * NS
LANES = 16
CH = 80


def _sc_degree(N_, E_):
    EPW = E_ // NW
    NCH = EPW // CH
    RPS = N_ // NS
    ZR = 125
    mesh = plsc.VectorSubcoreMesh(core_axis_name="c", subcore_axis_name="s")

    @functools.partial(
        pl.kernel,
        out_type=[jax.ShapeDtypeStruct((NC, N_, LANES), jnp.float32),
                  jax.ShapeDtypeStruct((NC, N_, LANES), jnp.float32)],
        mesh=mesh,
        compiler_params=pltpu.CompilerParams(use_tc_tiling_on_sc=False),
        scratch_types=[
            pltpu.VMEM((NCH, CH), jnp.int32),
            pltpu.VMEM((NCH, CH), jnp.int32),
            pltpu.VMEM((CH, LANES), jnp.float32),
            pltpu.VMEM((ZR, LANES), jnp.float32),
            pltpu.VMEM_SHARED((N_, LANES), jnp.float32),
            pltpu.VMEM_SHARED((N_, LANES), jnp.float32),
        ],
    )
    def deg_kernel(src_hbm, dst_hbm, osrc_hbm, odst_hbm,
                   srcv, dstv, ones_v, zeros_v, acc_s, acc_d):
        cid = lax.axis_index("c")
        sid = lax.axis_index("s")
        wid = sid * NC + cid

        @pl.loop(0, CH)
        def _(i):
            ones_v[i, :] = jnp.ones((LANES,), jnp.float32)

        @pl.loop(0, ZR)
        def _(i):
            zeros_v[i, :] = jnp.zeros((LANES,), jnp.float32)

        @pl.loop(0, RPS // ZR)
        def _(i):
            pltpu.sync_copy(zeros_v, acc_s.at[pl.ds(sid * RPS + i * ZR, ZR)])
            pltpu.sync_copy(zeros_v, acc_d.at[pl.ds(sid * RPS + i * ZR, ZR)])

        plsc.subcore_barrier()

        pltpu.sync_copy(src_hbm.at[wid], srcv)
        pltpu.sync_copy(dst_hbm.at[wid], dstv)

        @pl.loop(0, NCH)
        def _(j):
            pltpu.sync_copy(ones_v, acc_s.at[srcv.at[j]], add=True)
            pltpu.sync_copy(ones_v, acc_d.at[dstv.at[j]], add=True)

        plsc.subcore_barrier()

        @pl.when(sid == 0)
        def _():
            pltpu.sync_copy(acc_s, osrc_hbm.at[cid])

        @pl.when(sid == 1)
        def _():
            pltpu.sync_copy(acc_d, odst_hbm.at[cid])

    return deg_kernel


def _sc_aggregate(N_, E_, Dh):
    EPS = E_ // NS
    NCH = EPS // CH
    RPS = N_ // NS
    ZR = 25
    NBUF = 5
    mesh = plsc.VectorSubcoreMesh(core_axis_name="c", subcore_axis_name="s")

    @functools.partial(
        pl.kernel,
        out_type=jax.ShapeDtypeStruct((NC, N_, Dh), jnp.float32),
        mesh=mesh,
        compiler_params=pltpu.CompilerParams(use_tc_tiling_on_sc=False),
        scratch_types=[
            pltpu.VMEM((NCH, CH), jnp.int32),
            pltpu.VMEM((NCH, CH), jnp.int32),
            pltpu.VMEM((NBUF, CH, Dh), jnp.float32),
            pltpu.VMEM_SHARED((N_, Dh), jnp.float32),
            pltpu.SemaphoreType.DMA((NBUF,)),
            pltpu.SemaphoreType.DMA((NBUF,)),
        ],
    )
    def agg_kernel(h_hbm, src_hbm, dst_hbm, out_hbm,
                   srcv, dstv, rows_v, acc, gsem, ssem):
        cid = lax.axis_index("c")
        sid = lax.axis_index("s")
        h_c = h_hbm.at[cid]

        @pl.loop(0, ZR)
        def _(i):
            @pl.loop(0, Dh // LANES)
            def _(k):
                rows_v[0, i, pl.ds(k * LANES, LANES)] = jnp.zeros((LANES,), jnp.float32)

        @pl.loop(0, RPS // ZR)
        def _(i):
            pltpu.sync_copy(rows_v.at[0].at[pl.ds(0, ZR)],
                            acc.at[pl.ds(sid * RPS + i * ZR, ZR)])

        plsc.subcore_barrier()

        pltpu.sync_copy(src_hbm.at[sid], srcv)
        pltpu.sync_copy(dst_hbm.at[sid], dstv)

        for b in range(NBUF):
            pltpu.async_copy(h_c.at[srcv.at[b]], rows_v.at[b], gsem.at[b])

        @pl.loop(0, NCH, step=NBUF)
        def _(j):
            for b in range(NBUF):
                c = j + b
                pltpu.make_async_copy(h_c.at[srcv.at[c]], rows_v.at[b],
                                      gsem.at[b]).wait()
                pltpu.async_copy(rows_v.at[b], acc.at[dstv.at[c]], ssem.at[b],
                                 add=True)
            for b in range(NBUF):
                c = j + b
                c2 = j + NBUF + b
                pltpu.make_async_copy(rows_v.at[b], acc.at[dstv.at[c]],
                                      ssem.at[b]).wait()

                @pl.when(c2 < NCH)
                def _():
                    pltpu.async_copy(h_c.at[srcv.at[c2]], rows_v.at[b],
                                     gsem.at[b])

        plsc.subcore_barrier()

        @pl.when(sid == 0)
        def _():
            pltpu.sync_copy(acc, out_hbm.at[cid])

    return agg_kernel


def _norm(c0, c1):
    deg = c0[:, :1] + c1[:, :1]
    return lax.rsqrt(jnp.maximum(deg, 1.0))


def _tc_pre(feat, W, ds0, ds1, R=1000):
    N_, D = feat.shape
    Dout = W.shape[1]
    Dh = Dout // 2

    def body(f_r, w_r, d0_r, d1_r, o_r):
        h = jnp.dot(f_r[...], w_r[...], preferred_element_type=jnp.float32)
        h = h * _norm(d0_r, d1_r)
        o_r[0] = h[:, :Dh]
        o_r[1] = h[:, Dh:]

    return pl.pallas_call(
        body,
        grid=(N_ // R,),
        in_specs=[pl.BlockSpec((R, D), lambda i: (i, 0)),
                  pl.BlockSpec((D, Dout), lambda i: (0, 0)),
                  pl.BlockSpec((R, LANES), lambda i: (i, 0)),
                  pl.BlockSpec((R, LANES), lambda i: (i, 0))],
        out_specs=pl.BlockSpec((2, R, Dh), lambda i: (0, i, 0)),
        out_shape=jax.ShapeDtypeStruct((2, N_, Dh), jnp.float32),
    )(feat, W, ds0, ds1)


def _tc_mid(p, dd0, dd1, ds0, ds1, b, W, R=1000):
    _, N_, Dh_in = p.shape
    D = 2 * Dh_in
    Dout = W.shape[1]
    Dh = Dout // 2
    b2d = b.reshape(1, D)

    def body(p_r, dd0_r, dd1_r, ds0_r, ds1_r, b_r, w_r, o_r):
        agg = jnp.concatenate([p_r[0], p_r[1]], axis=1)
        h = agg * _norm(dd0_r, dd1_r) + b_r[...]
        h = jnp.maximum(h, 0.0)
        hw = jnp.dot(h, w_r[...], preferred_element_type=jnp.float32)
        hw = hw * _norm(ds0_r, ds1_r)
        o_r[0] = hw[:, :Dh]
        o_r[1] = hw[:, Dh:]

    return pl.pallas_call(
        body,
        grid=(N_ // R,),
        in_specs=[pl.BlockSpec((2, R, Dh_in), lambda i: (0, i, 0)),
                  pl.BlockSpec((R, LANES), lambda i: (i, 0)),
                  pl.BlockSpec((R, LANES), lambda i: (i, 0)),
                  pl.BlockSpec((R, LANES), lambda i: (i, 0)),
                  pl.BlockSpec((R, LANES), lambda i: (i, 0)),
                  pl.BlockSpec((1, D), lambda i: (0, 0)),
                  pl.BlockSpec((D, Dout), lambda i: (0, 0))],
        out_specs=pl.BlockSpec((2, R, Dh), lambda i: (0, i, 0)),
        out_shape=jax.ShapeDtypeStruct((2, N_, Dh), jnp.float32),
    )(p, dd0, dd1, ds0, ds1, b2d, W)


def _tc_post(p, dd0, dd1, b, R=1000):
    _, N_, Dh_in = p.shape
    D = 2 * Dh_in
    b2d = b.reshape(1, D)

    def body(p_r, dd0_r, dd1_r, b_r, o_r):
        agg = jnp.concatenate([p_r[0], p_r[1]], axis=1)
        o_r[...] = agg * _norm(dd0_r, dd1_r) + b_r[...]

    return pl.pallas_call(
        body,
        grid=(N_ // R,),
        in_specs=[pl.BlockSpec((2, R, Dh_in), lambda i: (0, i, 0)),
                  pl.BlockSpec((R, LANES), lambda i: (i, 0)),
                  pl.BlockSpec((R, LANES), lambda i: (i, 0)),
                  pl.BlockSpec((1, D), lambda i: (0, 0))],
        out_specs=pl.BlockSpec((R, D), lambda i: (i, 0)),
        out_shape=jax.ShapeDtypeStruct((N_, D), jnp.float32),
    )(p, dd0, dd1, b2d)


def kernel(feat, edge_index, W1, b1, W2, b2, W3, b3):
    N_, D_in = feat.shape
    E_ = edge_index.shape[1]
    HID_ = W2.shape[1]
    NCLS = W3.shape[1]
    D3 = 64

    src_w = edge_index[0].reshape(NW, (E_ // NW) // CH, CH)
    dst_w = edge_index[1].reshape(NW, (E_ // NW) // CH, CH)
    src_s = edge_index[0].reshape(NS, (E_ // NS) // CH, CH)
    dst_s = edge_index[1].reshape(NS, (E_ // NS) // CH, CH)

    W3p = jnp.zeros((HID_, D3), jnp.float32).at[:, :NCLS].set(W3)
    b3p = jnp.zeros((D3,), jnp.float32).at[:NCLS].set(b3)

    degs, degd = _sc_degree(N_, E_)(src_w, dst_w)
    ds0, ds1 = degs[0], degs[1]
    dd0, dd1 = degd[0], degd[1]

    agg_wide = _sc_aggregate(N_, E_, HID_ // 2)
    agg_narrow = _sc_aggregate(N_, E_, D3 // 2)

    h1 = _tc_pre(feat, W1, ds0, ds1)
    p1 = agg_wide(h1, src_s, dst_s)
    h2 = _tc_mid(p1, dd0, dd1, ds0, ds1, b1, W2)
    p2 = agg_wide(h2, src_s, dst_s)
    h3 = _tc_mid(p2, dd0, dd1, ds0, ds1, b2, W3p)
    p3 = agg_narrow(h3, src_s, dst_s)
    out = _tc_post(p3, dd0, dd1, b3p)
    return out[:, :NCLS]


# speedup vs baseline: 16.2475x; 1.0307x over previous
import functools

import jax
import jax.numpy as jnp
from jax import lax
from jax.experimental import pallas as pl
from jax.experimental.pallas import tpu as pltpu
from jax.experimental.pallas import tpu_sc as plsc

NC = 2
NS = 16
NW = NC * NS
LANES = 16
CH = 80


def _sc_degree(N_, E_):
    NCH = (E_ // NS) // CH
    NCHW = NCH // NC
    RPS = N_ // NS
    ZR = 125
    mesh = plsc.VectorSubcoreMesh(core_axis_name="c", subcore_axis_name="s")

    @functools.partial(
        pl.kernel,
        out_type=[jax.ShapeDtypeStruct((NC, N_, LANES), jnp.float32),
                  jax.ShapeDtypeStruct((NC, N_, LANES), jnp.float32)],
        mesh=mesh,
        compiler_params=pltpu.CompilerParams(use_tc_tiling_on_sc=False),
        scratch_types=[
            pltpu.VMEM((NCHW, CH), jnp.int32),
            pltpu.VMEM((NCHW, CH), jnp.int32),
            pltpu.VMEM((CH, LANES), jnp.float32),
            pltpu.VMEM((ZR, LANES), jnp.float32),
            pltpu.VMEM_SHARED((N_, LANES), jnp.float32),
            pltpu.VMEM_SHARED((N_, LANES), jnp.float32),
        ],
    )
    def deg_kernel(e_hbm, osrc_hbm, odst_hbm,
                   srcv, dstv, ones_v, zeros_v, acc_s, acc_d):
        cid = lax.axis_index("c")
        sid = lax.axis_index("s")

        @pl.loop(0, CH)
        def _(i):
            ones_v[i, :] = jnp.ones((LANES,), jnp.float32)

        @pl.loop(0, ZR)
        def _(i):
            zeros_v[i, :] = jnp.zeros((LANES,), jnp.float32)

        @pl.loop(0, RPS // ZR)
        def _(i):
            pltpu.sync_copy(zeros_v, acc_s.at[pl.ds(sid * RPS + i * ZR, ZR)])
            pltpu.sync_copy(zeros_v, acc_d.at[pl.ds(sid * RPS + i * ZR, ZR)])

        plsc.subcore_barrier()

        pltpu.sync_copy(e_hbm.at[0].at[sid].at[pl.ds(cid * NCHW, NCHW)], srcv)
        pltpu.sync_copy(e_hbm.at[1].at[sid].at[pl.ds(cid * NCHW, NCHW)], dstv)

        @pl.loop(0, NCHW)
        def _(j):
            pltpu.sync_copy(ones_v, acc_s.at[srcv.at[j]], add=True)
            pltpu.sync_copy(ones_v, acc_d.at[dstv.at[j]], add=True)

        plsc.subcore_barrier()

        @pl.when(sid == 0)
        def _():
            pltpu.sync_copy(acc_s, osrc_hbm.at[cid])

        @pl.when(sid == 1)
        def _():
            pltpu.sync_copy(acc_d, odst_hbm.at[cid])

    return deg_kernel


def _sc_aggregate(N_, E_, Dh):
    NCH = (E_ // NS) // CH
    RPS = N_ // NS
    ZR = 25
    NBUF = 5
    mesh = plsc.VectorSubcoreMesh(core_axis_name="c", subcore_axis_name="s")

    @functools.partial(
        pl.kernel,
        out_type=jax.ShapeDtypeStruct((NC, N_, Dh), jnp.float32),
        mesh=mesh,
        compiler_params=pltpu.CompilerParams(use_tc_tiling_on_sc=False),
        scratch_types=[
            pltpu.VMEM((NCH, CH), jnp.int32),
            pltpu.VMEM((NCH, CH), jnp.int32),
            pltpu.VMEM((NBUF, CH, Dh), jnp.float32),
            pltpu.VMEM_SHARED((N_, Dh), jnp.float32),
            pltpu.SemaphoreType.DMA((NBUF,)),
            pltpu.SemaphoreType.DMA((NBUF,)),
        ],
    )
    def agg_kernel(h_hbm, e_hbm, out_hbm,
                   srcv, dstv, rows_v, acc, gsem, ssem):
        cid = lax.axis_index("c")
        sid = lax.axis_index("s")
        h_c = h_hbm.at[cid]

        @pl.loop(0, ZR)
        def _(i):
            @pl.loop(0, Dh // LANES)
            def _(k):
                rows_v[0, i, pl.ds(k * LANES, LANES)] = jnp.zeros((LANES,), jnp.float32)

        @pl.loop(0, RPS // ZR)
        def _(i):
            pltpu.sync_copy(rows_v.at[0].at[pl.ds(0, ZR)],
                            acc.at[pl.ds(sid * RPS + i * ZR, ZR)])

        plsc.subcore_barrier()

        pltpu.sync_copy(e_hbm.at[0].at[sid], srcv)
        pltpu.sync_copy(e_hbm.at[1].at[sid], dstv)

        for b in range(NBUF):
            pltpu.async_copy(h_c.at[srcv.at[b]], rows_v.at[b], gsem.at[b])

        @pl.loop(0, NCH, step=NBUF)
        def _(j):
            for b in range(NBUF):
                c = j + b
                pltpu.make_async_copy(h_c.at[srcv.at[c]], rows_v.at[b],
                                      gsem.at[b]).wait()
                pltpu.async_copy(rows_v.at[b], acc.at[dstv.at[c]], ssem.at[b],
                                 add=True)
            for b in range(NBUF):
                c = j + b
                c2 = j + NBUF + b
                pltpu.make_async_copy(rows_v.at[b], acc.at[dstv.at[c]],
                                      ssem.at[b]).wait()

                @pl.when(c2 < NCH)
                def _():
                    pltpu.async_copy(h_c.at[srcv.at[c2]], rows_v.at[b],
                                     gsem.at[b])

        plsc.subcore_barrier()

        @pl.when(sid == 0)
        def _():
            pltpu.sync_copy(acc, out_hbm.at[cid])

    return agg_kernel


def _norm(c0, c1):
    deg = c0[:, :1] + c1[:, :1]
    return lax.rsqrt(jnp.maximum(deg, 1.0))


def _tc_matmul(x, W, R=2000):
    N_, D = x.shape
    Dout = W.shape[1]

    def body(x_r, w_r, o_r):
        o_r[...] = jnp.dot(x_r[...], w_r[...], preferred_element_type=jnp.float32)

    return pl.pallas_call(
        body,
        grid=(N_ // R,),
        in_specs=[pl.BlockSpec((R, D), lambda i: (i, 0)),
                  pl.BlockSpec((D, Dout), lambda i: (0, 0))],
        out_specs=pl.BlockSpec((R, Dout), lambda i: (i, 0)),
        out_shape=jax.ShapeDtypeStruct((N_, Dout), jnp.float32),
    )(x, W)


def _tc_scale_split(h, ds0, ds1, R=2000):
    N_, D = h.shape
    Dh = D // 2

    def body(h_r, d0_r, d1_r, o_r):
        h = h_r[...] * _norm(d0_r, d1_r)
        o_r[0] = h[:, :Dh]
        o_r[1] = h[:, Dh:]

    return pl.pallas_call(
        body,
        grid=(N_ // R,),
        in_specs=[pl.BlockSpec((R, D), lambda i: (i, 0)),
                  pl.BlockSpec((R, LANES), lambda i: (i, 0)),
                  pl.BlockSpec((R, LANES), lambda i: (i, 0))],
        out_specs=pl.BlockSpec((2, R, Dh), lambda i: (0, i, 0)),
        out_shape=jax.ShapeDtypeStruct((2, N_, Dh), jnp.float32),
    )(h, ds0, ds1)


def _tc_mid(p, dd0, dd1, ds0, ds1, b, W, R=2000):
    _, N_, Dh_in = p.shape
    D = 2 * Dh_in
    Dout = W.shape[1]
    Dh = Dout // 2
    b2d = b.reshape(1, D)

    def body(p_r, dd0_r, dd1_r, ds0_r, ds1_r, b_r, w_r, o_r):
        agg = jnp.concatenate([p_r[0], p_r[1]], axis=1)
        h = agg * _norm(dd0_r, dd1_r) + b_r[...]
        h = jnp.maximum(h, 0.0)
        hw = jnp.dot(h, w_r[...], preferred_element_type=jnp.float32)
        hw = hw * _norm(ds0_r, ds1_r)
        o_r[0] = hw[:, :Dh]
        o_r[1] = hw[:, Dh:]

    return pl.pallas_call(
        body,
        grid=(N_ // R,),
        in_specs=[pl.BlockSpec((2, R, Dh_in), lambda i: (0, i, 0)),
                  pl.BlockSpec((R, LANES), lambda i: (i, 0)),
                  pl.BlockSpec((R, LANES), lambda i: (i, 0)),
                  pl.BlockSpec((R, LANES), lambda i: (i, 0)),
                  pl.BlockSpec((R, LANES), lambda i: (i, 0)),
                  pl.BlockSpec((1, D), lambda i: (0, 0)),
                  pl.BlockSpec((D, Dout), lambda i: (0, 0))],
        out_specs=pl.BlockSpec((2, R, Dh), lambda i: (0, i, 0)),
        out_shape=jax.ShapeDtypeStruct((2, N_, Dh), jnp.float32),
    )(p, dd0, dd1, ds0, ds1, b2d, W)


def _tc_post(p, dd0, dd1, b, R=2000):
    _, N_, Dh_in = p.shape
    D = 2 * Dh_in
    b2d = b.reshape(1, D)

    def body(p_r, dd0_r, dd1_r, b_r, o_r):
        agg = jnp.concatenate([p_r[0], p_r[1]], axis=1)
        o_r[...] = agg * _norm(dd0_r, dd1_r) + b_r[...]

    return pl.pallas_call(
        body,
        grid=(N_ // R,),
        in_specs=[pl.BlockSpec((2, R, Dh_in), lambda i: (0, i, 0)),
                  pl.BlockSpec((R, LANES), lambda i: (i, 0)),
                  pl.BlockSpec((R, LANES), lambda i: (i, 0)),
                  pl.BlockSpec((1, D), lambda i: (0, 0))],
        out_specs=pl.BlockSpec((R, D), lambda i: (i, 0)),
        out_shape=jax.ShapeDtypeStruct((N_, D), jnp.float32),
    )(p, dd0, dd1, b2d)


def kernel(feat, edge_index, W1, b1, W2, b2, W3, b3):
    N_, D_in = feat.shape
    E_ = edge_index.shape[1]
    HID_ = W2.shape[1]
    NCLS = W3.shape[1]
    D3 = 64

    NCH = (E_ // NS) // CH
    e4 = edge_index.reshape(2, NS, NCH, CH)

    W3p = jnp.zeros((HID_, D3), jnp.float32).at[:, :NCLS].set(W3)
    b3p = jnp.zeros((D3,), jnp.float32).at[:NCLS].set(b3)

    degs, degd = _sc_degree(N_, E_)(e4)
    ds0, ds1 = degs[0], degs[1]
    dd0, dd1 = degd[0], degd[1]

    agg_wide = _sc_aggregate(N_, E_, HID_ // 2)
    agg_narrow = _sc_aggregate(N_, E_, D3 // 2)

    h1u = _tc_matmul(feat, W1)
    h1 = _tc_scale_split(h1u, ds0, ds1)
    p1 = agg_wide(h1, e4)
    h2 = _tc_mid(p1, dd0, dd1, ds0, ds1, b1, W2)
    p2 = agg_wide(h2, e4)
    h3 = _tc_mid(p2, dd0, dd1, ds0, ds1, b2, W3p)
    p3 = agg_narrow(h3, e4)
    out = _tc_post(p3, dd0, dd1, b3p)
    return out[:, :NCLS]
